# baseline (speedup 1.0000x reference)
import jax
import jax.numpy as jnp
from jax.experimental import pallas as pl
from jax.experimental.pallas import tpu as pltpu

_N = 50000
_HIDDEN = 128
_NUM_XFERS = 512
_BLOCK = 25000
_NBLK = _N // _BLOCK
_NEG = -1e30


def _critic_kernel(h_ref, w1t_ref, b1_ref, w2t_ref, b2_ref, g_ref,
                   node_ref, nlp_ref, ent_ref, val_ref,
                   m_ref, s_ref, t_ref, bv_ref, bi_ref, bq_ref):
    i = pl.program_id(0)

    @pl.when(i == 0)
    def _init():
        m_ref[0] = _NEG
        s_ref[0] = 0.0
        t_ref[0] = 0.0
        bv_ref[0] = _NEG
        bi_ref[0] = 0
        bq_ref[0] = 0.0

    hbT = h_ref[...].T
    relu1T = jnp.maximum(jnp.dot(w1t_ref[...], hbT) + b1_ref[...], 0.0)
    q = jnp.dot(w2t_ref[...], relu1T) + b2_ref[0, 0]

    lm = jnp.max(q)
    m_old = m_ref[0]
    m_new = jnp.maximum(m_old, lm)
    scale = jnp.exp(m_old - m_new)
    e = jnp.exp(q - m_new)
    s_ref[0] = s_ref[0] * scale + jnp.sum(e)
    t_ref[0] = t_ref[0] * scale + jnp.sum(e * q)
    m_ref[0] = m_new

    g = g_ref[0]
    pv = q + g
    lv = jnp.max(pv)
    iota = jax.lax.broadcasted_iota(jnp.int32, (1, _BLOCK), 1)
    li = jnp.min(jnp.where(pv == lv, iota, _N))
    lq = jnp.sum(jnp.where(iota == li, q, 0.0))

    @pl.when(lv > bv_ref[0])
    def _upd():
        bv_ref[0] = lv
        bi_ref[0] = i * _BLOCK + li
        bq_ref[0] = lq

    @pl.when(i == _NBLK - 1)
    def _fin():
        log_z = m_ref[0] + jnp.log(s_ref[0])
        node_ref[0, 0] = bi_ref[0]
        nlp_ref[0, 0] = bq_ref[0] - log_z
        ent_ref[0, 0] = log_z - t_ref[0] / s_ref[0]
        val_ref[0, 0] = bq_ref[0]


def _actor_kernel(node_sref, h_ref, w1_ref, b1_ref, w2_ref, b2_ref,
                  mask_ref, g_ref, xfer_ref, xlp_ref, xent_ref):
    del node_sref
    hrow = h_ref[0]
    relu1 = jnp.maximum(jnp.dot(hrow, w1_ref[...]) + b1_ref[...], 0.0)
    logits = jnp.dot(relu1, w2_ref[...]) + b2_ref[...]
    masked = jnp.where(mask_ref[...] > 0, logits, logits - 1e10)

    m = jnp.max(masked)
    e = jnp.exp(masked - m)
    s = jnp.sum(e)
    log_z = m + jnp.log(s)

    pv = masked + g_ref[...]
    lv = jnp.max(pv)
    iota = jax.lax.broadcasted_iota(jnp.int32, (1, _NUM_XFERS), 1)
    li = jnp.min(jnp.where(pv == lv, iota, _NUM_XFERS))
    lval = jnp.sum(jnp.where(iota == li, masked, 0.0))

    xfer_ref[0, 0] = li
    xlp_ref[0, 0] = lval - log_z
    xent_ref[0, 0] = log_z - jnp.sum(e * masked) / s


def kernel(h, mask, aW1, ab1, aW2, ab2, cW1, cb1, cW2, cb2):
    f32 = jnp.float32
    g_node = jax.random.gumbel(jax.random.key(42), (_N,), f32)
    g_xfer = jax.random.gumbel(jax.random.key(43), (_NUM_XFERS,), f32)

    node_o, nlp_o, ent_o, val_o = pl.pallas_call(
        _critic_kernel,
        grid=(_NBLK,),
        in_specs=[
            pl.BlockSpec((_BLOCK, _HIDDEN), lambda i: (i, 0)),
            pl.BlockSpec((_HIDDEN, _HIDDEN), lambda i: (0, 0)),
            pl.BlockSpec((_HIDDEN, 1), lambda i: (0, 0)),
            pl.BlockSpec((1, _HIDDEN), lambda i: (0, 0)),
            pl.BlockSpec(memory_space=pltpu.SMEM),
            pl.BlockSpec((1, 1, _BLOCK), lambda i: (i, 0, 0)),
        ],
        out_specs=[
            pl.BlockSpec(memory_space=pltpu.SMEM),
            pl.BlockSpec(memory_space=pltpu.SMEM),
            pl.BlockSpec(memory_space=pltpu.SMEM),
            pl.BlockSpec(memory_space=pltpu.SMEM),
        ],
        out_shape=[
            jax.ShapeDtypeStruct((1, 1), jnp.int32),
            jax.ShapeDtypeStruct((1, 1), f32),
            jax.ShapeDtypeStruct((1, 1), f32),
            jax.ShapeDtypeStruct((1, 1), f32),
        ],
        scratch_shapes=[
            pltpu.SMEM((1,), f32), pltpu.SMEM((1,), f32),
            pltpu.SMEM((1,), f32), pltpu.SMEM((1,), f32),
            pltpu.SMEM((1,), jnp.int32), pltpu.SMEM((1,), f32),
        ],
    )(h, cW1.T, cb1.reshape(_HIDDEN, 1), cW2.reshape(1, _HIDDEN),
      cb2.reshape(1, 1), g_node.reshape(_NBLK, 1, _BLOCK))

    node32 = node_o.reshape((1,))

    xfer_o, xlp_o, xent_o = pl.pallas_call(
        _actor_kernel,
        grid_spec=pltpu.PrefetchScalarGridSpec(
            num_scalar_prefetch=1,
            grid=(1,),
            in_specs=[
                pl.BlockSpec((1, 1, _HIDDEN), lambda i, n: (n[0], 0, 0)),
                pl.BlockSpec((_HIDDEN, _HIDDEN), lambda i, n: (0, 0)),
                pl.BlockSpec((1, _HIDDEN), lambda i, n: (0, 0)),
                pl.BlockSpec((_HIDDEN, _NUM_XFERS), lambda i, n: (0, 0)),
                pl.BlockSpec((1, _NUM_XFERS), lambda i, n: (0, 0)),
                pl.BlockSpec((1, _NUM_XFERS), lambda i, n: (0, 0)),
                pl.BlockSpec((1, _NUM_XFERS), lambda i, n: (0, 0)),
            ],
            out_specs=[
                pl.BlockSpec(memory_space=pltpu.SMEM),
                pl.BlockSpec(memory_space=pltpu.SMEM),
                pl.BlockSpec(memory_space=pltpu.SMEM),
            ],
        ),
        out_shape=[
            jax.ShapeDtypeStruct((1, 1), jnp.int32),
            jax.ShapeDtypeStruct((1, 1), f32),
            jax.ShapeDtypeStruct((1, 1), f32),
        ],
    )(node32, h.reshape(_N, 1, _HIDDEN), aW1, ab1.reshape(1, _HIDDEN),
      aW2, ab2.reshape(1, _NUM_XFERS),
      mask.astype(f32).reshape(1, _NUM_XFERS), g_xfer.reshape(1, _NUM_XFERS))

    return (node32[0], nlp_o[0, 0], ent_o[0, 0],
            xfer_o[0, 0], xlp_o[0, 0], xent_o[0, 0], val_o[0, 0])


# speedup vs baseline: 1.0380x; 1.0380x over previous
import jax
import jax.numpy as jnp
from jax.experimental import pallas as pl
from jax.experimental.pallas import tpu as pltpu

_N = 50000
_HIDDEN = 128
_NUM_XFERS = 512
_BLOCK = 10000
_NBLK = _N // _BLOCK
_NEG = -1e30


def _critic_kernel(h_ref, w1t_ref, b1_ref, w2t_ref, b2_ref, g_ref,
                   node_ref, nlp_ref, ent_ref, val_ref,
                   m_ref, s_ref, t_ref, bv_ref, bi_ref, bq_ref):
    i = pl.program_id(0)

    @pl.when(i == 0)
    def _init():
        m_ref[0] = _NEG
        s_ref[0] = 0.0
        t_ref[0] = 0.0
        bv_ref[0] = _NEG
        bi_ref[0] = 0
        bq_ref[0] = 0.0

    hbT = h_ref[...].T
    relu1T = jnp.maximum(jnp.dot(w1t_ref[...], hbT) + b1_ref[...], 0.0)
    q = jnp.dot(w2t_ref[...], relu1T) + b2_ref[0, 0]

    lm = jnp.max(q)
    m_old = m_ref[0]
    m_new = jnp.maximum(m_old, lm)
    scale = jnp.exp(m_old - m_new)
    e = jnp.exp(q - m_new)
    s_ref[0] = s_ref[0] * scale + jnp.sum(e)
    t_ref[0] = t_ref[0] * scale + jnp.sum(e * q)
    m_ref[0] = m_new

    g = g_ref[0]
    pv = q + g
    lv = jnp.max(pv)
    iota = jax.lax.broadcasted_iota(jnp.int32, (1, _BLOCK), 1)
    li = jnp.min(jnp.where(pv == lv, iota, _N))
    lq = jnp.sum(jnp.where(iota == li, q, 0.0))

    @pl.when(lv > bv_ref[0])
    def _upd():
        bv_ref[0] = lv
        bi_ref[0] = i * _BLOCK + li
        bq_ref[0] = lq

    @pl.when(i == _NBLK - 1)
    def _fin():
        log_z = m_ref[0] + jnp.log(s_ref[0])
        node_ref[0, 0] = bi_ref[0]
        nlp_ref[0, 0] = bq_ref[0] - log_z
        ent_ref[0, 0] = log_z - t_ref[0] / s_ref[0]
        val_ref[0, 0] = bq_ref[0]


def _actor_kernel(node_sref, h_ref, w1_ref, b1_ref, w2_ref, b2_ref,
                  mask_ref, g_ref, xfer_ref, xlp_ref, xent_ref):
    del node_sref
    hrow = h_ref[0]
    relu1 = jnp.maximum(jnp.dot(hrow, w1_ref[...]) + b1_ref[...], 0.0)
    logits = jnp.dot(relu1, w2_ref[...]) + b2_ref[...]
    masked = jnp.where(mask_ref[...] > 0, logits, logits - 1e10)

    m = jnp.max(masked)
    e = jnp.exp(masked - m)
    s = jnp.sum(e)
    log_z = m + jnp.log(s)

    pv = masked + g_ref[...]
    lv = jnp.max(pv)
    iota = jax.lax.broadcasted_iota(jnp.int32, (1, _NUM_XFERS), 1)
    li = jnp.min(jnp.where(pv == lv, iota, _NUM_XFERS))
    lval = jnp.sum(jnp.where(iota == li, masked, 0.0))

    xfer_ref[0, 0] = li
    xlp_ref[0, 0] = lval - log_z
    xent_ref[0, 0] = log_z - jnp.sum(e * masked) / s


def kernel(h, mask, aW1, ab1, aW2, ab2, cW1, cb1, cW2, cb2):
    f32 = jnp.float32
    g_node = jax.random.gumbel(jax.random.key(42), (_N,), f32)
    g_xfer = jax.random.gumbel(jax.random.key(43), (_NUM_XFERS,), f32)

    node_o, nlp_o, ent_o, val_o = pl.pallas_call(
        _critic_kernel,
        grid=(_NBLK,),
        in_specs=[
            pl.BlockSpec((_BLOCK, _HIDDEN), lambda i: (i, 0)),
            pl.BlockSpec((_HIDDEN, _HIDDEN), lambda i: (0, 0)),
            pl.BlockSpec((_HIDDEN, 1), lambda i: (0, 0)),
            pl.BlockSpec((1, _HIDDEN), lambda i: (0, 0)),
            pl.BlockSpec(memory_space=pltpu.SMEM),
            pl.BlockSpec((1, 1, _BLOCK), lambda i: (i, 0, 0)),
        ],
        out_specs=[
            pl.BlockSpec(memory_space=pltpu.SMEM),
            pl.BlockSpec(memory_space=pltpu.SMEM),
            pl.BlockSpec(memory_space=pltpu.SMEM),
            pl.BlockSpec(memory_space=pltpu.SMEM),
        ],
        out_shape=[
            jax.ShapeDtypeStruct((1, 1), jnp.int32),
            jax.ShapeDtypeStruct((1, 1), f32),
            jax.ShapeDtypeStruct((1, 1), f32),
            jax.ShapeDtypeStruct((1, 1), f32),
        ],
        scratch_shapes=[
            pltpu.SMEM((1,), f32), pltpu.SMEM((1,), f32),
            pltpu.SMEM((1,), f32), pltpu.SMEM((1,), f32),
            pltpu.SMEM((1,), jnp.int32), pltpu.SMEM((1,), f32),
        ],
    )(h, cW1.T, cb1.reshape(_HIDDEN, 1), cW2.reshape(1, _HIDDEN),
      cb2.reshape(1, 1), g_node.reshape(_NBLK, 1, _BLOCK))

    node32 = node_o.reshape((1,))

    xfer_o, xlp_o, xent_o = pl.pallas_call(
        _actor_kernel,
        grid_spec=pltpu.PrefetchScalarGridSpec(
            num_scalar_prefetch=1,
            grid=(1,),
            in_specs=[
                pl.BlockSpec((1, 1, _HIDDEN), lambda i, n: (n[0], 0, 0)),
                pl.BlockSpec((_HIDDEN, _HIDDEN), lambda i, n: (0, 0)),
                pl.BlockSpec((1, _HIDDEN), lambda i, n: (0, 0)),
                pl.BlockSpec((_HIDDEN, _NUM_XFERS), lambda i, n: (0, 0)),
                pl.BlockSpec((1, _NUM_XFERS), lambda i, n: (0, 0)),
                pl.BlockSpec((1, _NUM_XFERS), lambda i, n: (0, 0)),
                pl.BlockSpec((1, _NUM_XFERS), lambda i, n: (0, 0)),
            ],
            out_specs=[
                pl.BlockSpec(memory_space=pltpu.SMEM),
                pl.BlockSpec(memory_space=pltpu.SMEM),
                pl.BlockSpec(memory_space=pltpu.SMEM),
            ],
        ),
        out_shape=[
            jax.ShapeDtypeStruct((1, 1), jnp.int32),
            jax.ShapeDtypeStruct((1, 1), f32),
            jax.ShapeDtypeStruct((1, 1), f32),
        ],
    )(node32, h.reshape(_N, 1, _HIDDEN), aW1, ab1.reshape(1, _HIDDEN),
      aW2, ab2.reshape(1, _NUM_XFERS),
      mask.astype(f32).reshape(1, _NUM_XFERS), g_xfer.reshape(1, _NUM_XFERS))

    return (node32[0], nlp_o[0, 0], ent_o[0, 0],
            xfer_o[0, 0], xlp_o[0, 0], xent_o[0, 0], val_o[0, 0])


# speedup vs baseline: 1.2413x; 1.1958x over previous
import jax
import jax.numpy as jnp
from jax.experimental import pallas as pl
from jax.experimental.pallas import tpu as pltpu

_N = 50000
_HIDDEN = 128
_NUM_XFERS = 512
_BLOCK = 10000
_NBLK = _N // _BLOCK
_NEG = -1e30
_TINY = 1.1754943508222875e-38


def _threefry_gumbel(cnt, key2):
    K3 = (key2 ^ 0x1BD11BDA) & 0xFFFFFFFF
    u32 = lambda v: jnp.uint32(v & 0xFFFFFFFF)
    x0 = jnp.zeros_like(cnt)
    x1 = cnt + u32(key2)

    def rnd(x0, x1, r):
        x0 = x0 + x1
        x1 = (jax.lax.shift_left(x1, jnp.uint32(r))
              | jax.lax.shift_right_logical(x1, jnp.uint32(32 - r)))
        x1 = x0 ^ x1
        return x0, x1

    for r in (13, 15, 26, 6):
        x0, x1 = rnd(x0, x1, r)
    x0 = x0 + u32(key2); x1 = x1 + u32(K3 + 1)
    for r in (17, 29, 16, 24):
        x0, x1 = rnd(x0, x1, r)
    x0 = x0 + u32(K3); x1 = x1 + u32(2)
    for r in (13, 15, 26, 6):
        x0, x1 = rnd(x0, x1, r)
    x1 = x1 + u32(key2 + 3)
    for r in (17, 29, 16, 24):
        x0, x1 = rnd(x0, x1, r)
    x0 = x0 + u32(key2); x1 = x1 + u32(K3 + 4)
    for r in (13, 15, 26, 6):
        x0, x1 = rnd(x0, x1, r)
    x0 = x0 + u32(K3); x1 = x1 + u32(5)

    bits = x0 ^ x1
    fb = jax.lax.shift_right_logical(bits, jnp.uint32(9)) | jnp.uint32(0x3F800000)
    f = jax.lax.bitcast_convert_type(fb, jnp.float32) - jnp.float32(1.0)
    u = jnp.maximum(_TINY, f + _TINY)
    return -jnp.log(-jnp.log(u))


def _critic_kernel(h_ref, w1t_ref, b1_ref, w2t_ref, b2_ref,
                   node_ref, nlp_ref, ent_ref, val_ref,
                   m_ref, s_ref, t_ref, bv_ref, bi_ref, bq_ref):
    i = pl.program_id(0)

    @pl.when(i == 0)
    def _init():
        m_ref[0] = _NEG
        s_ref[0] = 0.0
        t_ref[0] = 0.0
        bv_ref[0] = _NEG
        bi_ref[0] = 0
        bq_ref[0] = 0.0

    hbT = h_ref[...].T
    relu1T = jnp.maximum(jnp.dot(w1t_ref[...], hbT) + b1_ref[...], 0.0)
    q = jnp.dot(w2t_ref[...], relu1T) + b2_ref[0, 0]

    lm = jnp.max(q)
    m_old = m_ref[0]
    m_new = jnp.maximum(m_old, lm)
    scale = jnp.exp(m_old - m_new)
    e = jnp.exp(q - m_new)
    s_ref[0] = s_ref[0] * scale + jnp.sum(e)
    t_ref[0] = t_ref[0] * scale + jnp.sum(e * q)
    m_ref[0] = m_new

    iota_u = jax.lax.broadcasted_iota(jnp.uint32, (1, _BLOCK), 1)
    g = _threefry_gumbel(iota_u + (i * _BLOCK).astype(jnp.uint32), 42)
    pv = q + g
    lv = jnp.max(pv)
    iota = jax.lax.broadcasted_iota(jnp.int32, (1, _BLOCK), 1)
    li = jnp.min(jnp.where(pv == lv, iota, _N))
    lq = jnp.sum(jnp.where(iota == li, q, 0.0))

    @pl.when(lv > bv_ref[0])
    def _upd():
        bv_ref[0] = lv
        bi_ref[0] = i * _BLOCK + li
        bq_ref[0] = lq

    @pl.when(i == _NBLK - 1)
    def _fin():
        log_z = m_ref[0] + jnp.log(s_ref[0])
        node_ref[0, 0] = bi_ref[0]
        nlp_ref[0, 0] = bq_ref[0] - log_z
        ent_ref[0, 0] = log_z - t_ref[0] / s_ref[0]
        val_ref[0, 0] = bq_ref[0]


def _actor_kernel(node_sref, h_ref, w1_ref, b1_ref, w2_ref, b2_ref,
                  mask_ref, xfer_ref, xlp_ref, xent_ref):
    del node_sref
    hrow = h_ref[0]
    relu1 = jnp.maximum(jnp.dot(hrow, w1_ref[...]) + b1_ref[...], 0.0)
    logits = jnp.dot(relu1, w2_ref[...]) + b2_ref[...]
    masked = jnp.where(mask_ref[...] > 0, logits, logits - 1e10)

    m = jnp.max(masked)
    e = jnp.exp(masked - m)
    s = jnp.sum(e)
    log_z = m + jnp.log(s)

    g = _threefry_gumbel(jax.lax.broadcasted_iota(jnp.uint32, (1, _NUM_XFERS), 1), 43)
    pv = masked + g
    lv = jnp.max(pv)
    iota = jax.lax.broadcasted_iota(jnp.int32, (1, _NUM_XFERS), 1)
    li = jnp.min(jnp.where(pv == lv, iota, _NUM_XFERS))
    lval = jnp.sum(jnp.where(iota == li, masked, 0.0))

    xfer_ref[0, 0] = li
    xlp_ref[0, 0] = lval - log_z
    xent_ref[0, 0] = log_z - jnp.sum(e * masked) / s


def kernel(h, mask, aW1, ab1, aW2, ab2, cW1, cb1, cW2, cb2):
    f32 = jnp.float32

    node_o, nlp_o, ent_o, val_o = pl.pallas_call(
        _critic_kernel,
        grid=(_NBLK,),
        in_specs=[
            pl.BlockSpec((_BLOCK, _HIDDEN), lambda i: (i, 0)),
            pl.BlockSpec((_HIDDEN, _HIDDEN), lambda i: (0, 0)),
            pl.BlockSpec((_HIDDEN, 1), lambda i: (0, 0)),
            pl.BlockSpec((1, _HIDDEN), lambda i: (0, 0)),
            pl.BlockSpec(memory_space=pltpu.SMEM),
        ],
        out_specs=[
            pl.BlockSpec(memory_space=pltpu.SMEM),
            pl.BlockSpec(memory_space=pltpu.SMEM),
            pl.BlockSpec(memory_space=pltpu.SMEM),
            pl.BlockSpec(memory_space=pltpu.SMEM),
        ],
        out_shape=[
            jax.ShapeDtypeStruct((1, 1), jnp.int32),
            jax.ShapeDtypeStruct((1, 1), f32),
            jax.ShapeDtypeStruct((1, 1), f32),
            jax.ShapeDtypeStruct((1, 1), f32),
        ],
        scratch_shapes=[
            pltpu.SMEM((1,), f32), pltpu.SMEM((1,), f32),
            pltpu.SMEM((1,), f32), pltpu.SMEM((1,), f32),
            pltpu.SMEM((1,), jnp.int32), pltpu.SMEM((1,), f32),
        ],
    )(h, cW1.T, cb1.reshape(_HIDDEN, 1), cW2.reshape(1, _HIDDEN),
      cb2.reshape(1, 1))

    node32 = node_o.reshape((1,))

    xfer_o, xlp_o, xent_o = pl.pallas_call(
        _actor_kernel,
        grid_spec=pltpu.PrefetchScalarGridSpec(
            num_scalar_prefetch=1,
            grid=(1,),
            in_specs=[
                pl.BlockSpec((1, 1, _HIDDEN), lambda i, n: (n[0], 0, 0)),
                pl.BlockSpec((_HIDDEN, _HIDDEN), lambda i, n: (0, 0)),
                pl.BlockSpec((1, _HIDDEN), lambda i, n: (0, 0)),
                pl.BlockSpec((_HIDDEN, _NUM_XFERS), lambda i, n: (0, 0)),
                pl.BlockSpec((1, _NUM_XFERS), lambda i, n: (0, 0)),
                pl.BlockSpec((1, _NUM_XFERS), lambda i, n: (0, 0)),
            ],
            out_specs=[
                pl.BlockSpec(memory_space=pltpu.SMEM),
                pl.BlockSpec(memory_space=pltpu.SMEM),
                pl.BlockSpec(memory_space=pltpu.SMEM),
            ],
        ),
        out_shape=[
            jax.ShapeDtypeStruct((1, 1), jnp.int32),
            jax.ShapeDtypeStruct((1, 1), f32),
            jax.ShapeDtypeStruct((1, 1), f32),
        ],
    )(node32, h.reshape(_N, 1, _HIDDEN), aW1, ab1.reshape(1, _HIDDEN),
      aW2, ab2.reshape(1, _NUM_XFERS),
      mask.astype(f32).reshape(1, _NUM_XFERS))

    return (node32[0], nlp_o[0, 0], ent_o[0, 0],
            xfer_o[0, 0], xlp_o[0, 0], xent_o[0, 0], val_o[0, 0])
